# baseline (speedup 1.0000x reference)
import functools

import jax
import jax.numpy as jnp
from jax import lax
from jax.experimental import pallas as pl
from jax.experimental.pallas import tpu as pltpu
from jax.experimental.pallas import tpu_sc as plsc

NC = 2
NS = 16
NW = NC * NS
VL = 16

ACH = 80
AGCH = 32
ANG = 4
ANCH = AGCH * ANG
EPWP = ANCH * ACH

CCH = 64
CNCH = EPWP // CCH

TRASH = 128
DEGPAD = 2304


def _sc_mesh():
    return plsc.VectorSubcoreMesh(core_axis_name="c", subcore_axis_name="s")


def _pass_a(src3, dst3, et3, n, r):
    nrp = n * r + DEGPAD
    deg_slice = nrp // NS
    kh = ACH // VL
    gsz = AGCH * ACH

    @functools.partial(
        pl.kernel,
        mesh=_sc_mesh(),
        out_type=(
            jax.ShapeDtypeStruct((NW * EPWP,), jnp.int32),
            jax.ShapeDtypeStruct((NW * EPWP,), jnp.float32),
        ),
        scratch_types=[
            pltpu.VMEM((AGCH, ACH), jnp.int32),
            pltpu.VMEM((AGCH, ACH), jnp.int32),
            pltpu.VMEM((AGCH, ACH), jnp.int32),
            pltpu.VMEM((AGCH, ACH), jnp.int32),
            pltpu.VMEM((gsz,), jnp.int32),
            pltpu.VMEM((gsz,), jnp.float32),
            pltpu.VMEM((ACH,), jnp.float32),
            pltpu.VMEM((AGCH, ACH), jnp.float32),
            pltpu.VMEM((deg_slice,), jnp.float32),
            pltpu.VMEM_SHARED((nrp,), jnp.float32),
            pltpu.SemaphoreType.DMA,
        ],
    )
    def k(src3_h, dst3_h, et3_h, gidx1_h, norm1_h,
          srcg, dstg, etg, didxg, gidxw, normw, ones, degv, zbuf, deg_sh, sem):
        c = lax.axis_index("c")
        s = lax.axis_index("s")
        w = c * NS + s

        zv = jnp.zeros((VL,), jnp.float32)

        def zi(i, carry):
            zbuf[pl.ds(i * VL, VL)] = zv
            return carry

        lax.fori_loop(0, deg_slice // VL, zi, 0)
        pltpu.sync_copy(zbuf, deg_sh.at[pl.ds(s * deg_slice, deg_slice)])
        ov = jnp.ones((VL,), jnp.float32)
        for kk in range(kh):
            ones[pl.ds(kk * VL, VL)] = ov
        plsc.subcore_barrier()

        def count_group(widx, g):
            pltpu.sync_copy(dst3_h.at[widx, pl.ds(g * AGCH, AGCH)], dstg)
            pltpu.sync_copy(et3_h.at[widx, pl.ds(g * AGCH, AGCH)], etg)

            def cj(j, carry):
                def ck(kk, carry2):
                    dd = dstg[j, pl.ds(kk * VL, VL)]
                    tt = etg[j, pl.ds(kk * VL, VL)]
                    didxg[j, pl.ds(kk * VL, VL)] = dd * r + tt
                    return carry2

                lax.fori_loop(0, kh, ck, 0)
                return carry

            lax.fori_loop(0, AGCH, cj, 0)

            def cf(j, carry):
                pltpu.async_copy(ones, deg_sh.at[didxg.at[j]], sem, add=True)
                return carry

            lax.fori_loop(0, AGCH, cf, 0)

            def cd(j, carry):
                pltpu.make_async_copy(
                    ones, deg_sh.at[didxg.at[j]], sem).wait()
                return carry

            lax.fori_loop(0, AGCH, cd, 0)

        def count_stripe(gi, carry):
            count_group((1 - c) * NS + s, gi)
            count_group(w, gi)
            return carry

        lax.fori_loop(0, ANG, count_stripe, 0)
        plsc.subcore_barrier()

        def pg(g, carry):
            pltpu.sync_copy(src3_h.at[w, pl.ds(g * AGCH, AGCH)], srcg)
            pltpu.sync_copy(dst3_h.at[w, pl.ds(g * AGCH, AGCH)], dstg)
            pltpu.sync_copy(et3_h.at[w, pl.ds(g * AGCH, AGCH)], etg)

            def pj(j, carry2):
                def pk(kk, carry3):
                    ss = srcg[j, pl.ds(kk * VL, VL)]
                    dd = dstg[j, pl.ds(kk * VL, VL)]
                    tt = etg[j, pl.ds(kk * VL, VL)]
                    gidxw[pl.ds(j * ACH + kk * VL, VL)] = tt * n + ss
                    didxg[j, pl.ds(kk * VL, VL)] = dd * r + tt
                    return carry3

                lax.fori_loop(0, kh, pk, 0)
                return carry2

            lax.fori_loop(0, AGCH, pj, 0)

            def gf(j, carry):
                pltpu.async_copy(deg_sh.at[didxg.at[j]], degv.at[j], sem)
                return carry

            lax.fori_loop(0, AGCH, gf, 0)

            def gd(j, carry):
                pltpu.make_async_copy(
                    deg_sh.at[didxg.at[j]], degv.at[j], sem).wait()
                return carry

            lax.fori_loop(0, AGCH, gd, 0)

            def nj(j, carry2):
                def nk(kk, carry3):
                    dv = degv[j, pl.ds(kk * VL, VL)]
                    normw[pl.ds(j * ACH + kk * VL, VL)] = (
                        1.0 / jnp.maximum(dv, 1.0))
                    return carry3

                lax.fori_loop(0, kh, nk, 0)
                return carry2

            lax.fori_loop(0, AGCH, nj, 0)
            pltpu.sync_copy(gidxw, gidx1_h.at[pl.ds(w * EPWP + g * gsz, gsz)])
            pltpu.sync_copy(normw, norm1_h.at[pl.ds(w * EPWP + g * gsz, gsz)])
            return carry

        lax.fori_loop(0, ANG, pg, 0)

    return k(src3, dst3, et3)


def _pass_c(table, gidx1, dst1, norm1, n, d):
    kd = d // VL
    nrows = n + TRASH

    @functools.partial(
        pl.kernel,
        mesh=_sc_mesh(),
        out_type=jax.ShapeDtypeStruct((NC, n, d), jnp.float32),
        scratch_types=[
            pltpu.VMEM((EPWP,), jnp.int32),
            pltpu.VMEM((EPWP,), jnp.int32),
            pltpu.VMEM((EPWP,), jnp.float32),
            pltpu.VMEM((CCH, d), jnp.float32),
            pltpu.VMEM((CCH, d), jnp.float32),
            pltpu.VMEM((CCH,), jnp.int32),
            pltpu.VMEM((CCH,), jnp.int32),
            pltpu.VMEM_SHARED((nrows, d), jnp.float32),
            pltpu.SemaphoreType.DMA,
            pltpu.SemaphoreType.DMA,
        ],
    )
    def k(table_h, gidx1_h, dst1_h, norm1_h, out_h,
          gidxw, dstw, normw, r0, r1, dc0, dc1, agg_sh, gsem, ssem):
        c = lax.axis_index("c")
        s = lax.axis_index("s")
        w = c * NS + s
        rbufs = (r0, r1)
        dbufs = (dc0, dc1)

        zv = jnp.zeros((VL,), jnp.float32)

        def ze(e, carry):
            def zk(kk, carry2):
                r0[e, pl.ds(kk * VL, VL)] = zv
                return carry2

            lax.fori_loop(0, kd, zk, 0)
            return carry

        lax.fori_loop(0, CCH, ze, 0)
        nps = n // NS
        npa = (nps // 8) * 8
        base = s * npa
        nfull = npa // CCH
        rem = npa - nfull * CCH
        for t in range(nfull):
            pltpu.sync_copy(r0, agg_sh.at[pl.ds(base + t * CCH, CCH)])
        if rem:
            pltpu.sync_copy(r0.at[pl.ds(0, rem)],
                            agg_sh.at[pl.ds(base + nfull * CCH, rem)])
        tail = nrows - NS * npa
        tfull = tail // CCH
        trem = tail - tfull * CCH

        @pl.when(s == NS - 1)
        def _():
            for t in range(tfull):
                pltpu.sync_copy(
                    r0, agg_sh.at[pl.ds(NS * npa + t * CCH, CCH)])
            if trem:
                pltpu.sync_copy(
                    r0.at[pl.ds(0, trem)],
                    agg_sh.at[pl.ds(NS * npa + tfull * CCH, trem)])
        plsc.subcore_barrier()

        pltpu.sync_copy(gidx1_h.at[pl.ds(w * EPWP, EPWP)], gidxw)
        pltpu.sync_copy(dst1_h.at[pl.ds(w * EPWP, EPWP)], dstw)
        pltpu.sync_copy(norm1_h.at[pl.ds(w * EPWP, EPWP)], normw)

        def issue_gather(j, rb):
            pltpu.async_copy(
                table_h.at[gidxw.at[pl.ds(j * CCH, CCH)]], rb, gsem)

        def chunk(j, b, first, last):
            rb = rbufs[b]
            db = dbufs[b]
            ob = rbufs[1 - b]
            odb = dbufs[1 - b]
            if first:
                @pl.when(j > 0)
                def _():
                    pltpu.make_async_copy(ob, agg_sh.at[odb], ssem).wait()
            else:
                pltpu.make_async_copy(ob, agg_sh.at[odb], ssem).wait()
            if not last:
                issue_gather(j + 1, ob)
            pltpu.make_async_copy(
                table_h.at[gidxw.at[pl.ds(j * CCH, CCH)]], rb, gsem).wait()
            for grp in range(CCH // VL):
                nv = normw[pl.ds(j * CCH + grp * VL, VL)]
                for l in range(VL):
                    e = grp * VL + l
                    nb = jnp.full((VL,), nv[l], jnp.float32)
                    for kk in range(kd):
                        rb[e, pl.ds(kk * VL, VL)] = (
                            rb[e, pl.ds(kk * VL, VL)] * nb)
            for grp in range(CCH // VL):
                db[pl.ds(grp * VL, VL)] = dstw[pl.ds(j * CCH + grp * VL, VL)]
            pltpu.async_copy(rb, agg_sh.at[db], ssem, add=True)

        issue_gather(0, r0)
        npair = CNCH // 2

        def pair(jj, carry):
            j = jj * 2
            chunk(j, 0, True, False)
            chunk(j + 1, 1, False, False)
            return carry

        lax.fori_loop(0, npair - 1, pair, 0)
        chunk(CNCH - 2, 0, True, False)
        chunk(CNCH - 1, 1, False, True)
        pltpu.make_async_copy(r1, agg_sh.at[dc1], ssem).wait()
        plsc.subcore_barrier()

        pltpu.sync_copy(agg_sh.at[pl.ds(base, npa)],
                        out_h.at[c, pl.ds(base, npa)])
        otail = n - NS * npa

        @pl.when(s == NS - 1)
        def _():
            pltpu.sync_copy(agg_sh.at[pl.ds(NS * npa, otail)],
                            out_h.at[c, pl.ds(NS * npa, otail)])

    return k(table, gidx1, dst1, norm1)


def _mm(x, w3, bn):
    n, d = x.shape
    rp = w3.shape[0]

    def body(x_ref, w_ref, o_ref):
        o_ref[0] = jnp.dot(x_ref[...].astype(jnp.bfloat16), w_ref[0],
                           preferred_element_type=jnp.float32)

    return pl.pallas_call(
        body,
        grid=(n // bn, rp),
        in_specs=[
            pl.BlockSpec((bn, d), lambda i, t: (i, 0)),
            pl.BlockSpec((1, d, d), lambda i, t: (t, 0, 0)),
        ],
        out_specs=pl.BlockSpec((1, bn, d), lambda i, t: (t, i, 0)),
        out_shape=jax.ShapeDtypeStruct((rp, n, d), jnp.float32),
    )(x, w3)


def _fused_layer2(agg, y0, bias0, w31, bn, r):
    nc, n, d = agg.shape
    rp = w31.shape[0]

    def body(agg_ref, rt_ref, b_ref, w_ref, h_ref, y_ref):
        h = jnp.maximum(agg_ref[0] + agg_ref[1] + rt_ref[0] + b_ref[...], 0.0)
        h_ref[...] = h
        y_ref[0] = jnp.dot(h.astype(jnp.bfloat16), w_ref[0],
                           preferred_element_type=jnp.float32)

    return pl.pallas_call(
        body,
        grid=(n // bn, rp),
        in_specs=[
            pl.BlockSpec((nc, bn, d), lambda i, t: (0, i, 0)),
            pl.BlockSpec((1, bn, d), lambda i, t: (r, i, 0)),
            pl.BlockSpec((1, d), lambda i, t: (0, 0)),
            pl.BlockSpec((1, d, d), lambda i, t: (t, 0, 0)),
        ],
        out_specs=[
            pl.BlockSpec((bn, d), lambda i, t: (i, 0)),
            pl.BlockSpec((1, bn, d), lambda i, t: (t, i, 0)),
        ],
        out_shape=[
            jax.ShapeDtypeStruct((n, d), jnp.float32),
            jax.ShapeDtypeStruct((rp, n, d), jnp.float32),
        ],
    )(agg, y0, bias0, w31)


def _final(agg, y1, bias1, h1, proj_w, proj_b, bn, r):
    nc, n, d = agg.shape
    grid = n // bn

    def body(agg_ref, rt_ref, b_ref, h1_ref, pw_ref, pb_ref, h2_ref, f_ref):
        h2 = jnp.maximum(agg_ref[0] + agg_ref[1] + rt_ref[0] + b_ref[...], 0.0)
        h2_ref[...] = h2
        pw = pw_ref[...]
        f_ref[...] = (
            jnp.dot(h1_ref[...], pw[:d], preferred_element_type=jnp.float32)
            + jnp.dot(h2, pw[d:], preferred_element_type=jnp.float32)
            + pb_ref[...]
        )

    return pl.pallas_call(
        body,
        grid=(grid,),
        in_specs=[
            pl.BlockSpec((nc, bn, d), lambda i: (0, i, 0)),
            pl.BlockSpec((1, bn, d), lambda i: (r, i, 0)),
            pl.BlockSpec((1, d), lambda i: (0, 0)),
            pl.BlockSpec((bn, d), lambda i: (i, 0)),
            pl.BlockSpec((2 * d, d), lambda i: (0, 0)),
            pl.BlockSpec((1, d), lambda i: (0, 0)),
        ],
        out_specs=[
            pl.BlockSpec((bn, d), lambda i: (i, 0)),
            pl.BlockSpec((bn, d), lambda i: (i, 0)),
        ],
        out_shape=[
            jax.ShapeDtypeStruct((n, d), jnp.float32),
            jax.ShapeDtypeStruct((n, d), jnp.float32),
        ],
    )(agg, y1, bias1, h1, proj_w, proj_b)


def kernel(node_feat, edge_index, edge_type, weight0, root0, bias0,
           weight1, root1, bias1, proj_w, proj_b):
    n, d = node_feat.shape
    r = weight0.shape[0]
    e = edge_type.shape[0]
    bn = 400

    pad = NW * EPWP - e
    idxp = jnp.arange(pad, dtype=jnp.int32)
    src = jnp.concatenate([edge_index[0], idxp % 1024])
    dst = jnp.concatenate([edge_index[1], n + (idxp % TRASH)])
    et = jnp.concatenate([edge_type, jnp.zeros((pad,), jnp.int32)])
    src3 = src.reshape(NW, ANCH, ACH)
    dst3 = dst.reshape(NW, ANCH, ACH)
    et3 = et.reshape(NW, ANCH, ACH)

    gidx1, norm1 = _pass_a(src3, dst3, et3, n, r)

    w30 = jnp.concatenate([weight0, root0[None]], axis=0).astype(jnp.bfloat16)
    w31 = jnp.concatenate([weight1, root1[None]], axis=0).astype(jnp.bfloat16)

    y0 = _mm(node_feat, w30, bn)
    agg0 = _pass_c(y0.reshape((r + 1) * n, d), gidx1, dst, norm1, n, d)
    h1, y1 = _fused_layer2(agg0, y0, bias0.reshape(1, d), w31, bn, r)
    agg1 = _pass_c(y1.reshape((r + 1) * n, d), gidx1, dst, norm1, n, d)
    h2, final = _final(agg1, y1, bias1.reshape(1, d), h1,
                       proj_w, proj_b.reshape(1, d), bn, r)
    return (final, h1, h2)


# speedup vs baseline: 1.8418x; 1.8418x over previous
import functools

import jax
import jax.numpy as jnp
from jax import lax
from jax.experimental import pallas as pl
from jax.experimental.pallas import tpu as pltpu
from jax.experimental.pallas import tpu_sc as plsc

NC = 2
NS = 16
NW = NC * NS
VL = 16

ACH = 80
AGCH = 32
ANG = 4
ANCH = AGCH * ANG
EPWP = ANCH * ACH

CCH = 64
CNCH = EPWP // CCH

TRASH = 128
DEGPAD = 2304


def _sc_mesh():
    return plsc.VectorSubcoreMesh(core_axis_name="c", subcore_axis_name="s")


def _pass_a(src3, dst3, et3, n, r):
    nrp = n * r + DEGPAD
    deg_slice = nrp // NS
    kh = ACH // VL
    gsz = AGCH * ACH

    @functools.partial(
        pl.kernel,
        mesh=_sc_mesh(),
        out_type=(
            jax.ShapeDtypeStruct((NW * EPWP,), jnp.int32),
            jax.ShapeDtypeStruct((NW * EPWP,), jnp.float32),
        ),
        scratch_types=[
            pltpu.VMEM((AGCH, ACH), jnp.int32),
            pltpu.VMEM((AGCH, ACH), jnp.int32),
            pltpu.VMEM((AGCH, ACH), jnp.int32),
            pltpu.VMEM((AGCH, ACH), jnp.int32),
            pltpu.VMEM((gsz,), jnp.int32),
            pltpu.VMEM((gsz,), jnp.float32),
            pltpu.VMEM((ACH,), jnp.float32),
            pltpu.VMEM((AGCH, ACH), jnp.float32),
            pltpu.VMEM((deg_slice,), jnp.float32),
            pltpu.VMEM_SHARED((nrp,), jnp.float32),
            pltpu.SemaphoreType.DMA,
        ],
    )
    def k(src3_h, dst3_h, et3_h, gidx1_h, norm1_h,
          srcg, dstg, etg, didxg, gidxw, normw, ones, degv, zbuf, deg_sh, sem):
        c = lax.axis_index("c")
        s = lax.axis_index("s")
        w = c * NS + s

        zv = jnp.zeros((VL,), jnp.float32)

        def zi(i, carry):
            zbuf[pl.ds(i * VL, VL)] = zv
            return carry

        lax.fori_loop(0, deg_slice // VL, zi, 0)
        pltpu.sync_copy(zbuf, deg_sh.at[pl.ds(s * deg_slice, deg_slice)])
        ov = jnp.ones((VL,), jnp.float32)
        for kk in range(kh):
            ones[pl.ds(kk * VL, VL)] = ov
        plsc.subcore_barrier()

        def count_group(widx, g):
            pltpu.sync_copy(dst3_h.at[widx, pl.ds(g * AGCH, AGCH)], dstg)
            pltpu.sync_copy(et3_h.at[widx, pl.ds(g * AGCH, AGCH)], etg)

            def cj(j, carry):
                def ck(kk, carry2):
                    dd = dstg[j, pl.ds(kk * VL, VL)]
                    tt = etg[j, pl.ds(kk * VL, VL)]
                    didxg[j, pl.ds(kk * VL, VL)] = dd * r + tt
                    return carry2

                lax.fori_loop(0, kh, ck, 0)
                return carry

            lax.fori_loop(0, AGCH, cj, 0)

            def cf(j, carry):
                pltpu.async_copy(ones, deg_sh.at[didxg.at[j]], sem, add=True)
                return carry

            lax.fori_loop(0, AGCH, cf, 0)

            def cd(j, carry):
                pltpu.make_async_copy(
                    ones, deg_sh.at[didxg.at[j]], sem).wait()
                return carry

            lax.fori_loop(0, AGCH, cd, 0)

        def count_stripe(gi, carry):
            count_group((1 - c) * NS + s, gi)
            count_group(w, gi)
            return carry

        lax.fori_loop(0, ANG, count_stripe, 0)
        plsc.subcore_barrier()

        def pg(g, carry):
            pltpu.sync_copy(src3_h.at[w, pl.ds(g * AGCH, AGCH)], srcg)
            pltpu.sync_copy(dst3_h.at[w, pl.ds(g * AGCH, AGCH)], dstg)
            pltpu.sync_copy(et3_h.at[w, pl.ds(g * AGCH, AGCH)], etg)

            def pj(j, carry2):
                def pk(kk, carry3):
                    ss = srcg[j, pl.ds(kk * VL, VL)]
                    dd = dstg[j, pl.ds(kk * VL, VL)]
                    tt = etg[j, pl.ds(kk * VL, VL)]
                    gidxw[pl.ds(j * ACH + kk * VL, VL)] = tt * n + ss
                    didxg[j, pl.ds(kk * VL, VL)] = dd * r + tt
                    return carry3

                lax.fori_loop(0, kh, pk, 0)
                return carry2

            lax.fori_loop(0, AGCH, pj, 0)

            def gf(j, carry):
                pltpu.async_copy(deg_sh.at[didxg.at[j]], degv.at[j], sem)
                return carry

            lax.fori_loop(0, AGCH, gf, 0)

            def gd(j, carry):
                pltpu.make_async_copy(
                    deg_sh.at[didxg.at[j]], degv.at[j], sem).wait()
                return carry

            lax.fori_loop(0, AGCH, gd, 0)

            def nj(j, carry2):
                def nk(kk, carry3):
                    dv = degv[j, pl.ds(kk * VL, VL)]
                    normw[pl.ds(j * ACH + kk * VL, VL)] = (
                        1.0 / jnp.maximum(dv, 1.0))
                    return carry3

                lax.fori_loop(0, kh, nk, 0)
                return carry2

            lax.fori_loop(0, AGCH, nj, 0)
            pltpu.sync_copy(gidxw, gidx1_h.at[pl.ds(w * EPWP + g * gsz, gsz)])
            pltpu.sync_copy(normw, norm1_h.at[pl.ds(w * EPWP + g * gsz, gsz)])
            return carry

        lax.fori_loop(0, ANG, pg, 0)

    return k(src3, dst3, et3)


def _pass_c(table, gidx1, dst1, norm1, n, d):
    kd = d // VL
    nrows = n + TRASH

    @functools.partial(
        pl.kernel,
        mesh=_sc_mesh(),
        out_type=jax.ShapeDtypeStruct((NC, n, d), jnp.float32),
        scratch_types=[
            pltpu.VMEM((EPWP,), jnp.int32),
            pltpu.VMEM((EPWP,), jnp.int32),
            pltpu.VMEM((EPWP,), jnp.float32),
            pltpu.VMEM((CCH, d), jnp.float32),
            pltpu.VMEM((CCH, d), jnp.float32),
            pltpu.VMEM((CCH,), jnp.int32),
            pltpu.VMEM((CCH,), jnp.int32),
            pltpu.VMEM_SHARED((nrows, d), jnp.float32),
            pltpu.SemaphoreType.DMA,
            pltpu.SemaphoreType.DMA,
        ],
    )
    def k(table_h, gidx1_h, dst1_h, norm1_h, out_h,
          gidxw, dstw, normw, r0, r1, dc0, dc1, agg_sh, gsem, ssem):
        c = lax.axis_index("c")
        s = lax.axis_index("s")
        w = c * NS + s
        rbufs = (r0, r1)
        dbufs = (dc0, dc1)

        zv = jnp.zeros((VL,), jnp.float32)

        def ze(e, carry):
            def zk(kk, carry2):
                r0[e, pl.ds(kk * VL, VL)] = zv
                return carry2

            lax.fori_loop(0, kd, zk, 0)
            return carry

        lax.fori_loop(0, CCH, ze, 0)
        nps = n // NS
        npa = (nps // 8) * 8
        base = s * npa
        nfull = npa // CCH
        rem = npa - nfull * CCH
        for t in range(nfull):
            pltpu.sync_copy(r0, agg_sh.at[pl.ds(base + t * CCH, CCH)])
        if rem:
            pltpu.sync_copy(r0.at[pl.ds(0, rem)],
                            agg_sh.at[pl.ds(base + nfull * CCH, rem)])
        tail = nrows - NS * npa
        tfull = tail // CCH
        trem = tail - tfull * CCH

        @pl.when(s == NS - 1)
        def _():
            for t in range(tfull):
                pltpu.sync_copy(
                    r0, agg_sh.at[pl.ds(NS * npa + t * CCH, CCH)])
            if trem:
                pltpu.sync_copy(
                    r0.at[pl.ds(0, trem)],
                    agg_sh.at[pl.ds(NS * npa + tfull * CCH, trem)])
        plsc.subcore_barrier()

        pltpu.sync_copy(gidx1_h.at[pl.ds(w * EPWP, EPWP)], gidxw)
        pltpu.sync_copy(dst1_h.at[pl.ds(w * EPWP, EPWP)], dstw)
        pltpu.sync_copy(norm1_h.at[pl.ds(w * EPWP, EPWP)], normw)

        def issue_gather(j, rb):
            pltpu.async_copy(
                table_h.at[gidxw.at[pl.ds(j * CCH, CCH)]], rb, gsem)

        def chunk(j, b, first, last):
            rb = rbufs[b]
            db = dbufs[b]
            ob = rbufs[1 - b]
            odb = dbufs[1 - b]
            if first:
                @pl.when(j > 0)
                def _():
                    pltpu.make_async_copy(ob, agg_sh.at[odb], ssem).wait()
            else:
                pltpu.make_async_copy(ob, agg_sh.at[odb], ssem).wait()
            if not last:
                issue_gather(j + 1, ob)
            pltpu.make_async_copy(
                table_h.at[gidxw.at[pl.ds(j * CCH, CCH)]], rb, gsem).wait()
            for grp in range(CCH // VL):
                nv = normw[pl.ds(j * CCH + grp * VL, VL)]
                for l in range(VL):
                    e = grp * VL + l
                    nb = jnp.full((VL,), nv[l], jnp.float32)
                    for kk in range(kd):
                        rb[e, pl.ds(kk * VL, VL)] = (
                            rb[e, pl.ds(kk * VL, VL)] * nb)
            for grp in range(CCH // VL):
                db[pl.ds(grp * VL, VL)] = dstw[pl.ds(j * CCH + grp * VL, VL)]
            pltpu.async_copy(rb, agg_sh.at[db], ssem, add=True)

        issue_gather(0, r0)
        npair = CNCH // 2

        def pair(jj, carry):
            j = jj * 2
            chunk(j, 0, True, False)
            chunk(j + 1, 1, False, False)
            return carry

        lax.fori_loop(0, npair - 1, pair, 0)
        chunk(CNCH - 2, 0, True, False)
        chunk(CNCH - 1, 1, False, True)
        pltpu.make_async_copy(r1, agg_sh.at[dc1], ssem).wait()
        plsc.subcore_barrier()

        pltpu.sync_copy(agg_sh.at[pl.ds(base, npa)],
                        out_h.at[c, pl.ds(base, npa)])
        otail = n - NS * npa

        @pl.when(s == NS - 1)
        def _():
            pltpu.sync_copy(agg_sh.at[pl.ds(NS * npa, otail)],
                            out_h.at[c, pl.ds(NS * npa, otail)])

    return k(table, gidx1, dst1, norm1)


def _mm(x, w3, bn):
    n, d = x.shape
    rp = w3.shape[0]

    def body(x_ref, w_ref, o_ref):
        xb = x_ref[...].astype(jnp.bfloat16)
        for t in range(rp):
            o_ref[t] = jnp.dot(xb, w_ref[t],
                               preferred_element_type=jnp.float32)

    return pl.pallas_call(
        body,
        grid=(n // bn,),
        in_specs=[
            pl.BlockSpec((bn, d), lambda i: (i, 0)),
            pl.BlockSpec((rp, d, d), lambda i: (0, 0, 0)),
        ],
        out_specs=pl.BlockSpec((rp, bn, d), lambda i: (0, i, 0)),
        out_shape=jax.ShapeDtypeStruct((rp, n, d), jnp.float32),
    )(x, w3)


def _fused_layer2(agg, y0, bias0, w31, bn, r):
    nc, n, d = agg.shape
    rp = w31.shape[0]

    def body(agg_ref, rt_ref, b_ref, w_ref, h_ref, y_ref):
        h = jnp.maximum(agg_ref[0] + agg_ref[1] + rt_ref[0] + b_ref[...], 0.0)
        h_ref[...] = h
        hb = h.astype(jnp.bfloat16)
        for t in range(rp):
            y_ref[t] = jnp.dot(hb, w_ref[t],
                               preferred_element_type=jnp.float32)

    return pl.pallas_call(
        body,
        grid=(n // bn,),
        in_specs=[
            pl.BlockSpec((nc, bn, d), lambda i: (0, i, 0)),
            pl.BlockSpec((1, bn, d), lambda i: (r, i, 0)),
            pl.BlockSpec((1, d), lambda i: (0, 0)),
            pl.BlockSpec((rp, d, d), lambda i: (0, 0, 0)),
        ],
        out_specs=[
            pl.BlockSpec((bn, d), lambda i: (i, 0)),
            pl.BlockSpec((rp, bn, d), lambda i: (0, i, 0)),
        ],
        out_shape=[
            jax.ShapeDtypeStruct((n, d), jnp.float32),
            jax.ShapeDtypeStruct((rp, n, d), jnp.float32),
        ],
    )(agg, y0, bias0, w31)


def _final(agg, y1, bias1, h1, proj_w, proj_b, bn, r):
    nc, n, d = agg.shape
    grid = n // bn

    def body(agg_ref, rt_ref, b_ref, h1_ref, pw_ref, pb_ref, h2_ref, f_ref):
        h2 = jnp.maximum(agg_ref[0] + agg_ref[1] + rt_ref[0] + b_ref[...], 0.0)
        h2_ref[...] = h2
        pw = pw_ref[...]
        f_ref[...] = (
            jnp.dot(h1_ref[...], pw[:d], preferred_element_type=jnp.float32)
            + jnp.dot(h2, pw[d:], preferred_element_type=jnp.float32)
            + pb_ref[...]
        )

    return pl.pallas_call(
        body,
        grid=(grid,),
        in_specs=[
            pl.BlockSpec((nc, bn, d), lambda i: (0, i, 0)),
            pl.BlockSpec((1, bn, d), lambda i: (r, i, 0)),
            pl.BlockSpec((1, d), lambda i: (0, 0)),
            pl.BlockSpec((bn, d), lambda i: (i, 0)),
            pl.BlockSpec((2 * d, d), lambda i: (0, 0)),
            pl.BlockSpec((1, d), lambda i: (0, 0)),
        ],
        out_specs=[
            pl.BlockSpec((bn, d), lambda i: (i, 0)),
            pl.BlockSpec((bn, d), lambda i: (i, 0)),
        ],
        out_shape=[
            jax.ShapeDtypeStruct((n, d), jnp.float32),
            jax.ShapeDtypeStruct((n, d), jnp.float32),
        ],
    )(agg, y1, bias1, h1, proj_w, proj_b)


def kernel(node_feat, edge_index, edge_type, weight0, root0, bias0,
           weight1, root1, bias1, proj_w, proj_b):
    n, d = node_feat.shape
    r = weight0.shape[0]
    e = edge_type.shape[0]
    bn = 400

    pad = NW * EPWP - e
    idxp = jnp.arange(pad, dtype=jnp.int32)
    src = jnp.concatenate([edge_index[0], idxp % 1024])
    dst = jnp.concatenate([edge_index[1], n + (idxp % TRASH)])
    et = jnp.concatenate([edge_type, jnp.zeros((pad,), jnp.int32)])
    src3 = src.reshape(NW, ANCH, ACH)
    dst3 = dst.reshape(NW, ANCH, ACH)
    et3 = et.reshape(NW, ANCH, ACH)

    gidx1, norm1 = _pass_a(src3, dst3, et3, n, r)

    w30 = jnp.concatenate([weight0, root0[None]], axis=0).astype(jnp.bfloat16)
    w31 = jnp.concatenate([weight1, root1[None]], axis=0).astype(jnp.bfloat16)

    y0 = _mm(node_feat, w30, bn)
    agg0 = _pass_c(y0.reshape((r + 1) * n, d), gidx1, dst, norm1, n, d)
    h1, y1 = _fused_layer2(agg0, y0, bias0.reshape(1, d), w31, bn, r)
    agg1 = _pass_c(y1.reshape((r + 1) * n, d), gidx1, dst, norm1, n, d)
    h2, final = _final(agg1, y1, bias1.reshape(1, d), h1,
                       proj_w, proj_b.reshape(1, d), bn, r)
    return (final, h1, h2)
